# baseline (speedup 1.0000x reference)
import jax
import jax.numpy as jnp
from jax.experimental import pallas as pl

BATCH = 16384
D = 3200
NUM_GROUPS = 10
SLICES_PER_GROUP = 5
SLICE_W = 32
GROUP_W = SLICES_PER_GROUP * SLICE_W

_TC_BLK = 1024
_NREFS = 3
_VIEW_W = 640
_CHUNKS_PER_VIEW = _VIEW_W // SLICE_W


def _tc_body(*refs):
    xs = refs[:_NREFS]
    out_refs = refs[_NREFS:]
    for g in range(NUM_GROUPS):
        parts = []
        for j in range(SLICES_PER_GROUP):
            chunk = j * NUM_GROUPS + g
            r, o = divmod(chunk, _CHUNKS_PER_VIEW)
            parts.append(xs[r][:, o * SLICE_W:(o + 1) * SLICE_W])
        out_refs[g][...] = jnp.concatenate(parts, axis=1)


_tc_call = pl.pallas_call(
    _tc_body,
    grid=(BATCH // _TC_BLK,),
    in_specs=[
        pl.BlockSpec((_TC_BLK, _VIEW_W), lambda i, c=c: (i, c))
        for c in range(_NREFS)
    ],
    out_specs=[
        pl.BlockSpec((_TC_BLK, GROUP_W), lambda i: (i, 0))
        for _ in range(NUM_GROUPS)
    ],
    out_shape=[
        jax.ShapeDtypeStruct((BATCH, GROUP_W), jnp.float32)
        for _ in range(NUM_GROUPS)
    ],
)


def kernel(input_tensor):
    return tuple(_tc_call(*([input_tensor] * _NREFS)))


# speedup vs baseline: 1.0174x; 1.0174x over previous
import jax
import jax.numpy as jnp
from jax.experimental import pallas as pl

BATCH = 16384
D = 3200
NUM_GROUPS = 10
SLICES_PER_GROUP = 5
SLICE_W = 32
GROUP_W = SLICES_PER_GROUP * SLICE_W

_TC_BLK = 1024
_VIEWS = [(640, 0), (640, 1), (128, 10), (128, 11), (128, 12)]
_NREFS = len(_VIEWS)


def _chunk_home(chunk):
    col = chunk * SLICE_W
    for r, (w, blk) in enumerate(_VIEWS):
        if w * blk <= col < w * (blk + 1):
            return r, (col - w * blk) // SLICE_W
    raise AssertionError(chunk)


def _tc_body(*refs):
    xs = refs[:_NREFS]
    out_refs = refs[_NREFS:]
    for g in range(NUM_GROUPS):
        parts = []
        for j in range(SLICES_PER_GROUP):
            r, o = _chunk_home(j * NUM_GROUPS + g)
            parts.append(xs[r][:, o * SLICE_W:(o + 1) * SLICE_W])
        out_refs[g][...] = jnp.concatenate(parts, axis=1)


_tc_call = pl.pallas_call(
    _tc_body,
    grid=(BATCH // _TC_BLK,),
    in_specs=[
        pl.BlockSpec((_TC_BLK, w), lambda i, blk=blk: (i, blk))
        for (w, blk) in _VIEWS
    ],
    out_specs=[
        pl.BlockSpec((_TC_BLK, GROUP_W), lambda i: (i, 0))
        for _ in range(NUM_GROUPS)
    ],
    out_shape=[
        jax.ShapeDtypeStruct((BATCH, GROUP_W), jnp.float32)
        for _ in range(NUM_GROUPS)
    ],
)


def kernel(input_tensor):
    return tuple(_tc_call(*([input_tensor] * _NREFS)))
